# baseline (speedup 1.0000x reference)
import jax
import jax.numpy as jnp
from jax.experimental import pallas as pl
from jax.experimental.pallas import tpu as pltpu

_N = 100000
_L = 256
_G = 512
_TILE = 4000
_NT = _N // _TILE
_W = 64


def _fused(lo_ref, hi_ref, z_ref, b_ref, w1_ref, b1_ref, w2_ref, b2_ref,
           mw1_ref, mb1_ref, mw2_ref, mb2_ref, mw3t_ref, mb3_ref,
           out_ref, s_ref, t_ref, m_ref):
    i = pl.program_id(0)

    @pl.when(i == 0)
    def _init():
        s_ref[:] = jnp.zeros_like(s_ref)
        t_ref[:] = jnp.zeros_like(t_ref)
        m_ref[0, 0] = -jnp.inf

    zb = z_ref[:].astype(jnp.bfloat16)
    h = jnp.tanh(jnp.dot(zb, w1_ref[:].astype(jnp.bfloat16),
                         preferred_element_type=jnp.float32)
                 + b1_ref[:])
    a = jnp.dot(h.astype(jnp.bfloat16), w2_ref[:].astype(jnp.bfloat16),
                preferred_element_type=jnp.float32) + b2_ref[0, 0]
    m_ref[0, 0] = jnp.maximum(m_ref[0, 0], jnp.max(a))
    e_row = jnp.exp(a).astype(jnp.bfloat16).reshape(1, _TILE)
    ids_row = b_ref[0]
    ones = jnp.ones((_TILE, 1), jnp.bfloat16)

    b0 = pl.multiple_of(jnp.minimum(lo_ref[i], _G - _W) & ~7, 8)

    @pl.when(hi_ref[i] - b0 < _W)
    def _narrow():
        oh = (jax.lax.broadcasted_iota(jnp.int32, (_W, _TILE), 0) + b0
              == ids_row).astype(jnp.bfloat16) * e_row
        s_ref[pl.ds(b0, _W), :] += jnp.dot(
            oh, zb, preferred_element_type=jnp.float32)
        t_ref[pl.ds(b0, _W), :] += jnp.dot(
            oh, ones, preferred_element_type=jnp.float32)

    @pl.when(hi_ref[i] - b0 >= _W)
    def _wide():
        oh = (jax.lax.broadcasted_iota(jnp.int32, (_G, _TILE), 0)
              == ids_row).astype(jnp.bfloat16) * e_row
        s_ref[:] += jnp.dot(oh, zb, preferred_element_type=jnp.float32)
        t_ref[:] += jnp.dot(oh, ones, preferred_element_type=jnp.float32)

    @pl.when(i == _NT - 1)
    def _final():
        eps = 1e-8 * jnp.exp(m_ref[0, 0])
        gz = s_ref[:] / (t_ref[:] + eps)
        x = jnp.maximum(jnp.dot(gz, mw1_ref[:],
                                preferred_element_type=jnp.float32)
                        + mb1_ref[:], 0.0)
        x = jnp.maximum(jnp.dot(x, mw2_ref[:],
                                preferred_element_type=jnp.float32)
                        + mb2_ref[:], 0.0)
        o = jnp.sum(x * mw3t_ref[:], axis=1, keepdims=True) + mb3_ref[0, 0]
        out_ref[:] = jax.nn.sigmoid(o)


def kernel(z, batch, att_w1, att_b1, att_w2, att_b2,
           mlp_w1, mlp_b1, mlp_w2, mlp_b2, mlp_w3, mlp_b3):
    b32 = batch.astype(jnp.int32)
    batch3d = b32.reshape(_NT, 1, _TILE)
    lo = b32[::_TILE]
    hi = b32[_TILE - 1::_TILE]
    full = lambda shape: pl.BlockSpec(shape, lambda i: (0, 0))
    smem = pl.BlockSpec(memory_space=pltpu.SMEM)
    out2d = pl.pallas_call(
        _fused,
        grid=(_NT,),
        in_specs=[
            smem,
            smem,
            pl.BlockSpec((_TILE, _L), lambda i: (i, 0)),
            pl.BlockSpec((1, 1, _TILE), lambda i: (i, 0, 0)),
            full((_L, _L)),
            full((1, _L)),
            full((_L, 1)),
            full((1, 1)),
            full((_L, 128)),
            full((1, 128)),
            full((128, 64)),
            full((1, 64)),
            full((1, 64)),
            full((1, 1)),
        ],
        out_specs=pl.BlockSpec((_G, 1), lambda i: (0, 0)),
        out_shape=jax.ShapeDtypeStruct((_G, 1), jnp.float32),
        scratch_shapes=[
            pltpu.VMEM((_G, _L), jnp.float32),
            pltpu.VMEM((_G, 1), jnp.float32),
            pltpu.SMEM((1, 1), jnp.float32),
        ],
    )(lo, hi, z, batch3d,
      att_w1, att_b1.reshape(1, _L), att_w2,
      att_b2.reshape(1, 1),
      mlp_w1, mlp_b1.reshape(1, 128), mlp_w2, mlp_b2.reshape(1, 64),
      mlp_w3.reshape(1, 64), mlp_b3.reshape(1, 1))
    return out2d.reshape(-1)


# speedup vs baseline: 1.2170x; 1.2170x over previous
import jax
import jax.numpy as jnp
from jax.experimental import pallas as pl
from jax.experimental.pallas import tpu as pltpu

_N = 100000
_L = 256
_G = 512
_TILE = 5000
_NT = _N // _TILE
_W = 64


def _fused(lo_ref, hi_ref, z_ref, b_ref, w1_ref, b1_ref, w2t_ref, b2_ref,
           mw1_ref, mb1_ref, mw2_ref, mb2_ref, mw3t_ref, mb3_ref,
           out_ref, s_ref, t_ref, m_ref):
    i = pl.program_id(0)

    @pl.when(i == 0)
    def _init():
        s_ref[:] = jnp.zeros_like(s_ref)
        t_ref[:] = jnp.zeros_like(t_ref)
        m_ref[0, 0] = -jnp.inf

    zb = z_ref[:].astype(jnp.bfloat16)
    h = jnp.tanh(jnp.dot(zb, w1_ref[:].astype(jnp.bfloat16),
                         preferred_element_type=jnp.float32)
                 + b1_ref[:])
    a = jnp.sum(h * w2t_ref[:], axis=1, keepdims=True) + b2_ref[0, 0]
    m_ref[0, 0] = jnp.maximum(m_ref[0, 0], jnp.max(a))
    e_row = jnp.exp(a).astype(jnp.bfloat16).reshape(1, _TILE)
    ids_row = b_ref[0]
    ones = jnp.ones((_TILE, 1), jnp.bfloat16)

    b0 = pl.multiple_of(jnp.minimum(lo_ref[i], _G - _W) & ~7, 8)

    @pl.when(hi_ref[i] - b0 < _W)
    def _narrow():
        oh = (jax.lax.broadcasted_iota(jnp.int32, (_W, _TILE), 0) + b0
              == ids_row).astype(jnp.bfloat16) * e_row
        s_ref[pl.ds(b0, _W), :] += jnp.dot(
            oh, zb, preferred_element_type=jnp.float32)
        t_ref[pl.ds(b0, _W), :] += jnp.dot(
            oh, ones, preferred_element_type=jnp.float32)

    @pl.when(hi_ref[i] - b0 >= _W)
    def _wide():
        oh = (jax.lax.broadcasted_iota(jnp.int32, (_G, _TILE), 0)
              == ids_row).astype(jnp.bfloat16) * e_row
        s_ref[:] += jnp.dot(oh, zb, preferred_element_type=jnp.float32)
        t_ref[:] += jnp.dot(oh, ones, preferred_element_type=jnp.float32)

    @pl.when(i == _NT - 1)
    def _final():
        eps = 1e-8 * jnp.exp(m_ref[0, 0])
        gz = s_ref[:] / (t_ref[:] + eps)
        x = jnp.maximum(jnp.dot(gz, mw1_ref[:],
                                preferred_element_type=jnp.float32)
                        + mb1_ref[:], 0.0)
        x = jnp.maximum(jnp.dot(x, mw2_ref[:],
                                preferred_element_type=jnp.float32)
                        + mb2_ref[:], 0.0)
        o = jnp.sum(x * mw3t_ref[:], axis=1, keepdims=True) + mb3_ref[0, 0]
        out_ref[:] = jax.nn.sigmoid(o)


def kernel(z, batch, att_w1, att_b1, att_w2, att_b2,
           mlp_w1, mlp_b1, mlp_w2, mlp_b2, mlp_w3, mlp_b3):
    b32 = batch.astype(jnp.int32)
    batch3d = b32.reshape(_NT, 1, _TILE)
    lo = b32[::_TILE]
    hi = b32[_TILE - 1::_TILE]
    full = lambda shape: pl.BlockSpec(shape, lambda i: (0, 0))
    smem = pl.BlockSpec(memory_space=pltpu.SMEM)
    out2d = pl.pallas_call(
        _fused,
        grid=(_NT,),
        in_specs=[
            smem,
            smem,
            pl.BlockSpec((_TILE, _L), lambda i: (i, 0)),
            pl.BlockSpec((1, 1, _TILE), lambda i: (i, 0, 0)),
            full((_L, _L)),
            full((1, _L)),
            full((1, _L)),
            full((1, 1)),
            full((_L, 128)),
            full((1, 128)),
            full((128, 64)),
            full((1, 64)),
            full((1, 64)),
            full((1, 1)),
        ],
        out_specs=pl.BlockSpec((_G, 1), lambda i: (0, 0)),
        out_shape=jax.ShapeDtypeStruct((_G, 1), jnp.float32),
        scratch_shapes=[
            pltpu.VMEM((_G, _L), jnp.float32),
            pltpu.VMEM((_G, 1), jnp.float32),
            pltpu.SMEM((1, 1), jnp.float32),
        ],
    )(lo, hi, z, batch3d,
      att_w1, att_b1.reshape(1, _L), att_w2.reshape(1, _L),
      att_b2.reshape(1, 1),
      mlp_w1, mlp_b1.reshape(1, 128), mlp_w2, mlp_b2.reshape(1, 64),
      mlp_w3.reshape(1, 64), mlp_b3.reshape(1, 1))
    return out2d.reshape(-1)


# speedup vs baseline: 1.3217x; 1.0861x over previous
import jax
import jax.numpy as jnp
from jax.experimental import pallas as pl
from jax.experimental.pallas import tpu as pltpu

_N = 100000
_L = 256
_G = 512
_TILE = 10000
_NT = _N // _TILE
_W = 64


def _fused(lo_ref, hi_ref, z_ref, b_ref, w1_ref, b1_ref, w2t_ref, b2_ref,
           mw1_ref, mb1_ref, mw2_ref, mb2_ref, mw3t_ref, mb3_ref,
           out_ref, s_ref, t_ref, m_ref):
    i = pl.program_id(0)

    @pl.when(i == 0)
    def _init():
        s_ref[:] = jnp.zeros_like(s_ref)
        t_ref[:] = jnp.zeros_like(t_ref)
        m_ref[0, 0] = -jnp.inf

    zb = z_ref[:].astype(jnp.bfloat16)
    h = jnp.tanh(jnp.dot(zb, w1_ref[:].astype(jnp.bfloat16),
                         preferred_element_type=jnp.float32)
                 + b1_ref[:])
    a = jnp.sum(h * w2t_ref[:], axis=1, keepdims=True) + b2_ref[0, 0]
    m_ref[0, 0] = jnp.maximum(m_ref[0, 0], jnp.max(a))
    e_row = jnp.exp(a).astype(jnp.bfloat16).reshape(1, _TILE)
    ids_row = b_ref[0]
    ones = jnp.ones((_TILE, 1), jnp.bfloat16)

    b0 = pl.multiple_of(jnp.minimum(lo_ref[i], _G - _W) & ~7, 8)

    @pl.when(hi_ref[i] - b0 < _W)
    def _narrow():
        oh = (jax.lax.broadcasted_iota(jnp.int32, (_W, _TILE), 0) + b0
              == ids_row).astype(jnp.bfloat16) * e_row
        s_ref[pl.ds(b0, _W), :] += jnp.dot(
            oh, zb, preferred_element_type=jnp.float32)
        t_ref[pl.ds(b0, _W), :] += jnp.dot(
            oh, ones, preferred_element_type=jnp.float32)

    @pl.when(hi_ref[i] - b0 >= _W)
    def _wide():
        oh = (jax.lax.broadcasted_iota(jnp.int32, (_G, _TILE), 0)
              == ids_row).astype(jnp.bfloat16) * e_row
        s_ref[:] += jnp.dot(oh, zb, preferred_element_type=jnp.float32)
        t_ref[:] += jnp.dot(oh, ones, preferred_element_type=jnp.float32)

    @pl.when(i == _NT - 1)
    def _final():
        eps = 1e-8 * jnp.exp(m_ref[0, 0])
        gz = s_ref[:] / (t_ref[:] + eps)
        x = jnp.maximum(jnp.dot(gz, mw1_ref[:],
                                preferred_element_type=jnp.float32)
                        + mb1_ref[:], 0.0)
        x = jnp.maximum(jnp.dot(x, mw2_ref[:],
                                preferred_element_type=jnp.float32)
                        + mb2_ref[:], 0.0)
        o = jnp.sum(x * mw3t_ref[:], axis=1, keepdims=True) + mb3_ref[0, 0]
        out_ref[:] = jax.nn.sigmoid(o)


def kernel(z, batch, att_w1, att_b1, att_w2, att_b2,
           mlp_w1, mlp_b1, mlp_w2, mlp_b2, mlp_w3, mlp_b3):
    b32 = batch.astype(jnp.int32)
    batch3d = b32.reshape(_NT, 1, _TILE)
    lo = b32[::_TILE]
    hi = b32[_TILE - 1::_TILE]
    full = lambda shape: pl.BlockSpec(shape, lambda i: (0, 0))
    smem = pl.BlockSpec(memory_space=pltpu.SMEM)
    out2d = pl.pallas_call(
        _fused,
        grid=(_NT,),
        in_specs=[
            smem,
            smem,
            pl.BlockSpec((_TILE, _L), lambda i: (i, 0)),
            pl.BlockSpec((1, 1, _TILE), lambda i: (i, 0, 0)),
            full((_L, _L)),
            full((1, _L)),
            full((1, _L)),
            full((1, 1)),
            full((_L, 128)),
            full((1, 128)),
            full((128, 64)),
            full((1, 64)),
            full((1, 64)),
            full((1, 1)),
        ],
        out_specs=pl.BlockSpec((_G, 1), lambda i: (0, 0)),
        out_shape=jax.ShapeDtypeStruct((_G, 1), jnp.float32),
        scratch_shapes=[
            pltpu.VMEM((_G, _L), jnp.float32),
            pltpu.VMEM((_G, 1), jnp.float32),
            pltpu.SMEM((1, 1), jnp.float32),
        ],
    )(lo, hi, z, batch3d,
      att_w1, att_b1.reshape(1, _L), att_w2.reshape(1, _L),
      att_b2.reshape(1, 1),
      mlp_w1, mlp_b1.reshape(1, 128), mlp_w2, mlp_b2.reshape(1, 64),
      mlp_w3.reshape(1, 64), mlp_b3.reshape(1, 1))
    return out2d.reshape(-1)
